# baseline (speedup 1.0000x reference)
import functools

import jax
import jax.numpy as jnp
import numpy as np
from jax.experimental import pallas as pl
from jax.experimental.pallas import tpu as pltpu
from jax.experimental.pallas import tpu_sc as plsc

_C = 128
_G = 128
_H = 4
_DH = _C // _H
_FFN = 4 * _C
_BG = 8


def _spread_bits(v):
    v = v & 0x3FF
    v = (v | (v << 16)) & 0x030000FF
    v = (v | (v << 8)) & 0x0300F00F
    v = (v | (v << 4)) & 0x030C30C3
    v = (v | (v << 2)) & 0x09249249
    return v


def _curve_encode(coors):
    b = coors[:, 0]
    x = _spread_bits(coors[:, 1])
    y = _spread_bits(coors[:, 2])
    z = _spread_bits(coors[:, 3])
    code = x | (y << 1) | (z << 2)
    return (b << 32) + code


def _sc_gather(table, idx):
    n_idx = idx.shape[0]
    d = table.shape[1]
    win = 128
    mesh = plsc.VectorSubcoreMesh(core_axis_name="c", subcore_axis_name="s")

    @pl.kernel(
        out_type=jax.ShapeDtypeStruct((n_idx, d), table.dtype),
        mesh=mesh,
    )
    def gather_kernel(x_hbm, i_hbm, o_hbm):
        def body(i_vmem, o_vmem):
            pltpu.sync_copy(x_hbm.at[i_vmem.at[0]], o_vmem)

        pltpu.emit_pipeline(
            body,
            grid=(n_idx // win,),
            in_specs=[pl.BlockSpec((1, win), lambda i: (0, i))],
            out_specs=[pl.BlockSpec((win, d), lambda i: (i, 0))],
            core_axis_name=("c", "s"),
            dimension_semantics=(pltpu.PARALLEL,),
        )(i_hbm, o_hbm)

    return gather_kernel(table, idx.reshape(1, n_idx))


def _ln_fast(x, g, b, j):
    f32 = jnp.float32
    xx = jnp.concatenate([x, x * x], axis=1)
    s = jnp.dot(xx, j, preferred_element_type=f32)
    inv_c = 1.0 / _C
    mu = s[:, 0:1] * inv_c
    ex2 = s[:, 1:2] * inv_c
    var = jnp.maximum(ex2 - mu * mu, 0.0)
    return (x - mu) * jax.lax.rsqrt(var + 1e-5) * g + b


def _pemb_kernel(x_ref, p_ref, wp01_ref, a_ref, t_ref):
    f32 = jnp.float32
    pe = jnp.dot(p_ref[...], wp01_ref[...], preferred_element_type=f32)
    a_ref[...] = x_ref[...] + pe[:, :_C]
    t_ref[...] = pe[:, _C:]


def _pemb(x, pts16, wp01):
    n = x.shape[0]
    rows = 4096
    return pl.pallas_call(
        _pemb_kernel,
        grid=(n // rows,),
        in_specs=[
            pl.BlockSpec((rows, _C), lambda i: (i, 0)),
            pl.BlockSpec((rows, 16), lambda i: (i, 0)),
            pl.BlockSpec((16, 2 * _C), lambda i: (0, 0)),
        ],
        out_specs=[pl.BlockSpec((rows, _C), lambda i: (i, 0)),
                   pl.BlockSpec((rows, _C), lambda i: (i, 0))],
        out_shape=[jax.ShapeDtypeStruct((n, _C), jnp.float32),
                   jax.ShapeDtypeStruct((n, _C), jnp.float32)],
        compiler_params=pltpu.CompilerParams(
            dimension_semantics=("parallel",)),
    )(x, pts16, wp01)


def _vox_kernel(*refs, add_input):
    f32 = jnp.float32
    bf = jnp.bfloat16
    if add_input:
        x_ref, t_ref, wqkv_ref, wo_ref, w1_ref, w2_ref, ln_ref, j_ref, \
            o_ref = refs
        x = x_ref[...] + t_ref[...]
    else:
        x_ref, wqkv_ref, wo_ref, w1_ref, w2_ref, ln_ref, j_ref, o_ref = refs
        x = x_ref[...]
    g1 = ln_ref[0:1, :]
    b1 = ln_ref[1:2, :]
    g2 = ln_ref[2:3, :]
    b2 = ln_ref[3:4, :]
    j = j_ref[...]
    h = _ln_fast(x, g1, b1, j).astype(bf)
    qkv = jnp.dot(h, wqkv_ref[...],
                  preferred_element_type=f32).astype(bf)
    rows = x.shape[0]
    qb = qkv[:, 0:_C]
    kt = jnp.transpose(qkv[:, _C:2 * _C])
    vb = qkv[:, 2 * _C:3 * _C]
    sub_iota = jax.lax.broadcasted_iota(jnp.int32, (_C, _G), 0) // _DH
    lane_iota = jax.lax.broadcasted_iota(jnp.int32, (_G, _C), 1) // _DH
    r_iota = jax.lax.broadcasted_iota(jnp.int32, (_H * _G, _C), 0) // _G
    l_iota = jax.lax.broadcasted_iota(jnp.int32, (_H * _G, _C), 1)
    ones_part = (l_iota == r_iota).astype(bf)
    groups = []
    for gi in range(rows // _G):
        r0 = gi * _G
        qg = qb[r0:r0 + _G, :]
        ktg = kt[:, r0:r0 + _G]
        vg = vb[r0:r0 + _G, :]
        kbd = jnp.concatenate(
            [jnp.where(sub_iota == hh, ktg, 0) for hh in range(_H)], axis=1)
        s_wide = jnp.dot(qg, kbd, preferred_element_type=f32)
        es = []
        for hh in range(_H):
            sh = s_wide[:, hh * _G:(hh + 1) * _G]
            mh = jnp.max(sh, axis=-1, keepdims=True)
            es.append(jnp.exp(sh - mh))
        e = jnp.concatenate(es, axis=1).astype(bf)
        vbd = jnp.concatenate(
            [jnp.concatenate(
                [jnp.where(lane_iota == hh, vg, 0) for hh in range(_H)],
                axis=0),
             ones_part], axis=1)
        osum = jnp.dot(e, vbd, preferred_element_type=f32)
        og = jnp.concatenate(
            [osum[:, hh * _DH:(hh + 1) * _DH] / osum[:, _C + hh:_C + hh + 1]
             for hh in range(_H)], axis=1)
        groups.append(og)
    att = jnp.concatenate(groups, axis=0).astype(bf)
    x = x + jnp.dot(att, wo_ref[...], preferred_element_type=f32)
    h2 = _ln_fast(x, g2, b2, j).astype(bf)
    f = jnp.maximum(jnp.dot(h2, w1_ref[...], preferred_element_type=f32),
                    0.0).astype(bf)
    x = x + jnp.dot(f, w2_ref[...], preferred_element_type=f32)
    o_ref[...] = x


def _voxformer(x, prm, tail=None):
    n = x.shape[0]
    rows = _BG * _G
    bf = jnp.bfloat16
    inv_sqrt_dh = np.float32(1.0 / np.sqrt(_DH))
    wqkv = jnp.concatenate(
        [prm['Wq'] * inv_sqrt_dh, prm['Wk'], prm['Wv']], axis=1).astype(bf)
    lnp = jnp.pad(
        jnp.stack([prm['g1'], prm['b1'], prm['g2'], prm['b2']]),
        ((0, 4), (0, 0)))
    j = jnp.zeros((2 * _C, _C), jnp.float32)
    j = j.at[: _C, 0].set(1.0).at[_C:, 1].set(1.0)
    add_input = tail is not None
    body = functools.partial(_vox_kernel, add_input=add_input)
    row_spec = pl.BlockSpec((rows, _C), lambda i: (i, 0))
    in_specs = [row_spec] + ([row_spec] if add_input else []) + [
        pl.BlockSpec((_C, 3 * _C), lambda i: (0, 0)),
        pl.BlockSpec((_C, _C), lambda i: (0, 0)),
        pl.BlockSpec((_C, _FFN), lambda i: (0, 0)),
        pl.BlockSpec((_FFN, _C), lambda i: (0, 0)),
        pl.BlockSpec((8, _C), lambda i: (0, 0)),
        pl.BlockSpec((2 * _C, _C), lambda i: (0, 0)),
    ]
    args = [x] + ([tail] if add_input else []) + [
        wqkv, prm['Wo'].astype(bf), prm['W1'].astype(bf),
        prm['W2'].astype(bf), lnp, j]
    return pl.pallas_call(
        body,
        grid=(n // rows,),
        in_specs=in_specs,
        out_specs=row_spec,
        out_shape=jax.ShapeDtypeStruct((n, _C), jnp.float32),
        compiler_params=pltpu.CompilerParams(
            dimension_semantics=("parallel",)),
    )(*args)


def kernel(voxel_numbers, voxel_coords, voxel_features, point_coords, params):
    n = voxel_features.shape[0]
    codes1 = _curve_encode(voxel_coords)
    codes2 = _curve_encode(voxel_coords[:, jnp.array([0, 3, 2, 1])])
    ind1 = jnp.argsort(codes1)
    ind2 = jnp.argsort(codes2)
    inv1 = jnp.zeros((n,), ind1.dtype).at[ind1].set(
        jnp.arange(n, dtype=ind1.dtype))
    ind12 = inv1[ind2]

    pts16 = jnp.pad(point_coords, ((0, 0), (0, 13)))
    wp01 = jnp.pad(
        jnp.concatenate([params[0]['Wp'], params[1]['Wp']], axis=1),
        ((0, 13), (0, 0)))
    a0, p1 = _pemb(voxel_features, pts16, wp01)
    x1 = _sc_gather(a0, ind1)
    gp2 = _sc_gather(p1, ind2)
    y0 = _voxformer(x1, params[0])
    x2 = _sc_gather(y0, ind12)
    return _voxformer(x2, params[1], tail=gp2)


# speedup vs baseline: 1.0379x; 1.0379x over previous
import functools

import jax
import jax.numpy as jnp
import numpy as np
from jax.experimental import pallas as pl
from jax.experimental.pallas import tpu as pltpu
from jax.experimental.pallas import tpu_sc as plsc

_C = 128
_G = 128
_H = 4
_DH = _C // _H
_FFN = 4 * _C
_BG = 8


def _spread_bits(v):
    v = v & 0x3FF
    v = (v | (v << 16)) & 0x030000FF
    v = (v | (v << 8)) & 0x0300F00F
    v = (v | (v << 4)) & 0x030C30C3
    v = (v | (v << 2)) & 0x09249249
    return v


def _curve_encode(coors):
    b = coors[:, 0]
    x = _spread_bits(coors[:, 1])
    y = _spread_bits(coors[:, 2])
    z = _spread_bits(coors[:, 3])
    code = x | (y << 1) | (z << 2)
    return (b << 32) + code


def _sc_gather(table, idx):
    n_idx = idx.shape[0]
    d = table.shape[1]
    win = 128
    mesh = plsc.VectorSubcoreMesh(core_axis_name="c", subcore_axis_name="s")

    @pl.kernel(
        out_type=jax.ShapeDtypeStruct((n_idx, d), table.dtype),
        mesh=mesh,
    )
    def gather_kernel(x_hbm, i_hbm, o_hbm):
        def body(i_vmem, o_vmem):
            pltpu.sync_copy(x_hbm.at[i_vmem.at[0]], o_vmem)

        pltpu.emit_pipeline(
            body,
            grid=(n_idx // win,),
            in_specs=[pl.BlockSpec((1, win), lambda i: (0, i))],
            out_specs=[pl.BlockSpec((win, d), lambda i: (i, 0))],
            core_axis_name=("c", "s"),
            dimension_semantics=(pltpu.PARALLEL,),
        )(i_hbm, o_hbm)

    return gather_kernel(table, idx.reshape(1, n_idx))


def _ln_fast(x, g, b, j):
    f32 = jnp.float32
    xx = jnp.concatenate([x, x * x], axis=1)
    s = jnp.dot(xx, j, preferred_element_type=f32)
    inv_c = 1.0 / _C
    mu = s[:, 0:1] * inv_c
    ex2 = s[:, 1:2] * inv_c
    var = jnp.maximum(ex2 - mu * mu, 0.0)
    return (x - mu) * jax.lax.rsqrt(var + 1e-5) * g + b


def _vox_kernel(x_ref, p_ref, wp_ref, wqkv_ref, wo_ref, w1_ref, w2_ref,
                ln_ref, j_ref, o_ref):
    f32 = jnp.float32
    bf = jnp.bfloat16
    x = x_ref[...] + jnp.dot(p_ref[...], wp_ref[...],
                             preferred_element_type=f32)
    g1 = ln_ref[0:1, :]
    b1 = ln_ref[1:2, :]
    g2 = ln_ref[2:3, :]
    b2 = ln_ref[3:4, :]
    j = j_ref[...]
    h = _ln_fast(x, g1, b1, j).astype(bf)
    qkv = jnp.dot(h, wqkv_ref[...],
                  preferred_element_type=f32).astype(bf)
    rows = x.shape[0]
    qb = qkv[:, 0:_C]
    kt = jnp.transpose(qkv[:, _C:2 * _C])
    vb = qkv[:, 2 * _C:3 * _C]
    sub_iota = jax.lax.broadcasted_iota(jnp.int32, (_C, _G), 0) // _DH
    lane_iota = jax.lax.broadcasted_iota(jnp.int32, (_G, _C), 1) // _DH
    r_iota = jax.lax.broadcasted_iota(jnp.int32, (_H * _G, _C), 0) // _G
    l_iota = jax.lax.broadcasted_iota(jnp.int32, (_H * _G, _C), 1)
    ones_part = (l_iota == r_iota).astype(bf)
    groups = []
    for gi in range(rows // _G):
        r0 = gi * _G
        qg = qb[r0:r0 + _G, :]
        ktg = kt[:, r0:r0 + _G]
        vg = vb[r0:r0 + _G, :]
        kbd = jnp.concatenate(
            [jnp.where(sub_iota == hh, ktg, 0) for hh in range(_H)], axis=1)
        s_wide = jnp.dot(qg, kbd, preferred_element_type=f32)
        es = []
        for hh in range(_H):
            sh = s_wide[:, hh * _G:(hh + 1) * _G]
            mh = jnp.max(sh, axis=-1, keepdims=True)
            es.append(jnp.exp(sh - mh))
        e = jnp.concatenate(es, axis=1).astype(bf)
        vbd = jnp.concatenate(
            [jnp.concatenate(
                [jnp.where(lane_iota == hh, vg, 0) for hh in range(_H)],
                axis=0),
             ones_part], axis=1)
        osum = jnp.dot(e, vbd, preferred_element_type=f32)
        og = jnp.concatenate(
            [osum[:, hh * _DH:(hh + 1) * _DH] / osum[:, _C + hh:_C + hh + 1]
             for hh in range(_H)], axis=1)
        groups.append(og)
    att = jnp.concatenate(groups, axis=0).astype(bf)
    x = x + jnp.dot(att, wo_ref[...], preferred_element_type=f32)
    h2 = _ln_fast(x, g2, b2, j).astype(bf)
    f = jnp.maximum(jnp.dot(h2, w1_ref[...], preferred_element_type=f32),
                    0.0).astype(bf)
    x = x + jnp.dot(f, w2_ref[...], preferred_element_type=f32)
    o_ref[...] = x


def _voxformer(x, pts_pad, prm):
    n = x.shape[0]
    rows = _BG * _G
    bf = jnp.bfloat16
    inv_sqrt_dh = np.float32(1.0 / np.sqrt(_DH))
    wqkv = jnp.concatenate(
        [prm['Wq'] * inv_sqrt_dh, prm['Wk'], prm['Wv']], axis=1).astype(bf)
    wp = jnp.pad(prm['Wp'], ((0, 13), (0, 0)))
    lnp = jnp.pad(
        jnp.stack([prm['g1'], prm['b1'], prm['g2'], prm['b2']]),
        ((0, 4), (0, 0)))
    j = jnp.zeros((2 * _C, _C), jnp.float32)
    j = j.at[: _C, 0].set(1.0).at[_C:, 1].set(1.0)
    return pl.pallas_call(
        _vox_kernel,
        grid=(n // rows,),
        in_specs=[
            pl.BlockSpec((rows, _C), lambda i: (i, 0)),
            pl.BlockSpec((rows, 16), lambda i: (i, 0)),
            pl.BlockSpec((16, _C), lambda i: (0, 0)),
            pl.BlockSpec((_C, 3 * _C), lambda i: (0, 0)),
            pl.BlockSpec((_C, _C), lambda i: (0, 0)),
            pl.BlockSpec((_C, _FFN), lambda i: (0, 0)),
            pl.BlockSpec((_FFN, _C), lambda i: (0, 0)),
            pl.BlockSpec((8, _C), lambda i: (0, 0)),
            pl.BlockSpec((2 * _C, _C), lambda i: (0, 0)),
        ],
        out_specs=pl.BlockSpec((rows, _C), lambda i: (i, 0)),
        out_shape=jax.ShapeDtypeStruct((n, _C), jnp.float32),
        compiler_params=pltpu.CompilerParams(
            dimension_semantics=("parallel",)),
    )(x, pts_pad, wp, wqkv, prm['Wo'].astype(bf), prm['W1'].astype(bf),
      prm['W2'].astype(bf), lnp, j)


def kernel(voxel_numbers, voxel_coords, voxel_features, point_coords, params):
    n = voxel_features.shape[0]
    codes1 = _curve_encode(voxel_coords)
    codes2 = _curve_encode(voxel_coords[:, jnp.array([0, 3, 2, 1])])
    ind1 = jnp.argsort(codes1)
    ind2 = jnp.argsort(codes2)
    inv1 = jnp.zeros((n,), ind1.dtype).at[ind1].set(
        jnp.arange(n, dtype=ind1.dtype))
    ind12 = inv1[ind2]

    pts_pad = jnp.pad(point_coords, ((0, 0), (0, _C - 3)))

    x1 = _sc_gather(voxel_features, ind1)
    p1 = _sc_gather(pts_pad, ind1)[:, :16]
    p2 = _sc_gather(pts_pad, ind2)[:, :16]
    y0 = _voxformer(x1, p1, params[0])
    x2 = _sc_gather(y0, ind12)
    return _voxformer(x2, p2, params[1])


# speedup vs baseline: 1.1960x; 1.1524x over previous
import functools

import jax
import jax.numpy as jnp
import numpy as np
from jax.experimental import pallas as pl
from jax.experimental.pallas import tpu as pltpu
from jax.experimental.pallas import tpu_sc as plsc

_C = 128
_G = 128
_H = 4
_DH = _C // _H
_FFN = 4 * _C
_BG = 16


def _spread_bits(v):
    v = v & 0x3FF
    v = (v | (v << 16)) & 0x030000FF
    v = (v | (v << 8)) & 0x0300F00F
    v = (v | (v << 4)) & 0x030C30C3
    v = (v | (v << 2)) & 0x09249249
    return v


def _curve_encode(coors):
    b = coors[:, 0]
    x = _spread_bits(coors[:, 1])
    y = _spread_bits(coors[:, 2])
    z = _spread_bits(coors[:, 3])
    code = x | (y << 1) | (z << 2)
    return (b << 32) + code


def _sc_gather(table, idx):
    n_idx = idx.shape[0]
    d = table.shape[1]
    win = 128
    mesh = plsc.VectorSubcoreMesh(core_axis_name="c", subcore_axis_name="s")

    @pl.kernel(
        out_type=jax.ShapeDtypeStruct((n_idx, d), table.dtype),
        mesh=mesh,
    )
    def gather_kernel(x_hbm, i_hbm, o_hbm):
        def body(i_vmem, o_vmem):
            pltpu.sync_copy(x_hbm.at[i_vmem.at[0]], o_vmem)

        pltpu.emit_pipeline(
            body,
            grid=(n_idx // win,),
            in_specs=[pl.BlockSpec((1, win), lambda i: (0, i))],
            out_specs=[pl.BlockSpec((win, d), lambda i: (i, 0))],
            core_axis_name=("c", "s"),
            dimension_semantics=(pltpu.PARALLEL,),
        )(i_hbm, o_hbm)

    return gather_kernel(table, idx.reshape(1, n_idx))


def _ln_fast(x, g, b, j):
    f32 = jnp.float32
    xx = jnp.concatenate([x, x * x], axis=1)
    s = jnp.dot(xx, j, preferred_element_type=f32)
    inv_c = 1.0 / _C
    mu = s[:, 0:1] * inv_c
    ex2 = s[:, 1:2] * inv_c
    var = jnp.maximum(ex2 - mu * mu, 0.0)
    return (x - mu) * jax.lax.rsqrt(var + 1e-5) * g + b


def _vox_kernel(x_ref, p_ref, wp_ref, wqkv_ref, wo_ref, w1_ref, w2_ref,
                ln_ref, j_ref, o_ref):
    f32 = jnp.float32
    bf = jnp.bfloat16
    x = x_ref[...] + jnp.dot(p_ref[...], wp_ref[...],
                             preferred_element_type=f32)
    g1 = ln_ref[0:1, :]
    b1 = ln_ref[1:2, :]
    g2 = ln_ref[2:3, :]
    b2 = ln_ref[3:4, :]
    j = j_ref[...]
    h = _ln_fast(x, g1, b1, j).astype(bf)
    qkv = jnp.dot(h, wqkv_ref[...],
                  preferred_element_type=f32).astype(bf)
    rows = x.shape[0]
    qb = qkv[:, 0:_C]
    kt = jnp.transpose(qkv[:, _C:2 * _C])
    vb = qkv[:, 2 * _C:3 * _C]
    sub_iota = jax.lax.broadcasted_iota(jnp.int32, (_C, _G), 0) // _DH
    lane_iota = jax.lax.broadcasted_iota(jnp.int32, (_G, _C), 1) // _DH
    r_iota = jax.lax.broadcasted_iota(jnp.int32, (_H * _G, _C), 0) // _G
    l_iota = jax.lax.broadcasted_iota(jnp.int32, (_H * _G, _C), 1)
    ones_part = (l_iota == r_iota).astype(bf)
    groups = []
    for gi in range(rows // _G):
        r0 = gi * _G
        qg = qb[r0:r0 + _G, :]
        ktg = kt[:, r0:r0 + _G]
        vg = vb[r0:r0 + _G, :]
        kbd = jnp.concatenate(
            [jnp.where(sub_iota == hh, ktg, 0) for hh in range(_H)], axis=1)
        s_wide = jnp.dot(qg, kbd, preferred_element_type=f32)
        es = []
        for hh in range(_H):
            sh = s_wide[:, hh * _G:(hh + 1) * _G]
            mh = jnp.max(sh, axis=-1, keepdims=True)
            es.append(jnp.exp(sh - mh))
        e = jnp.concatenate(es, axis=1).astype(bf)
        vbd = jnp.concatenate(
            [jnp.concatenate(
                [jnp.where(lane_iota == hh, vg, 0) for hh in range(_H)],
                axis=0),
             ones_part], axis=1)
        osum = jnp.dot(e, vbd, preferred_element_type=f32)
        og = jnp.concatenate(
            [osum[:, hh * _DH:(hh + 1) * _DH] / osum[:, _C + hh:_C + hh + 1]
             for hh in range(_H)], axis=1)
        groups.append(og)
    att = jnp.concatenate(groups, axis=0).astype(bf)
    x = x + jnp.dot(att, wo_ref[...], preferred_element_type=f32)
    h2 = _ln_fast(x, g2, b2, j).astype(bf)
    f = jnp.maximum(jnp.dot(h2, w1_ref[...], preferred_element_type=f32),
                    0.0).astype(bf)
    x = x + jnp.dot(f, w2_ref[...], preferred_element_type=f32)
    o_ref[...] = x


def _voxformer(x, pts_pad, prm):
    n = x.shape[0]
    rows = _BG * _G
    bf = jnp.bfloat16
    inv_sqrt_dh = np.float32(1.0 / np.sqrt(_DH))
    wqkv = jnp.concatenate(
        [prm['Wq'] * inv_sqrt_dh, prm['Wk'], prm['Wv']], axis=1).astype(bf)
    wp = jnp.pad(prm['Wp'], ((0, 13), (0, 0)))
    lnp = jnp.pad(
        jnp.stack([prm['g1'], prm['b1'], prm['g2'], prm['b2']]),
        ((0, 4), (0, 0)))
    j = jnp.zeros((2 * _C, _C), jnp.float32)
    j = j.at[: _C, 0].set(1.0).at[_C:, 1].set(1.0)
    return pl.pallas_call(
        _vox_kernel,
        grid=(n // rows,),
        in_specs=[
            pl.BlockSpec((rows, _C), lambda i: (i, 0)),
            pl.BlockSpec((rows, 16), lambda i: (i, 0)),
            pl.BlockSpec((16, _C), lambda i: (0, 0)),
            pl.BlockSpec((_C, 3 * _C), lambda i: (0, 0)),
            pl.BlockSpec((_C, _C), lambda i: (0, 0)),
            pl.BlockSpec((_C, _FFN), lambda i: (0, 0)),
            pl.BlockSpec((_FFN, _C), lambda i: (0, 0)),
            pl.BlockSpec((8, _C), lambda i: (0, 0)),
            pl.BlockSpec((2 * _C, _C), lambda i: (0, 0)),
        ],
        out_specs=pl.BlockSpec((rows, _C), lambda i: (i, 0)),
        out_shape=jax.ShapeDtypeStruct((n, _C), jnp.float32),
        compiler_params=pltpu.CompilerParams(
            dimension_semantics=("parallel",)),
    )(x, pts_pad, wp, wqkv, prm['Wo'].astype(bf), prm['W1'].astype(bf),
      prm['W2'].astype(bf), lnp, j)


def kernel(voxel_numbers, voxel_coords, voxel_features, point_coords, params):
    n = voxel_features.shape[0]
    codes1 = _curve_encode(voxel_coords)
    codes2 = _curve_encode(voxel_coords[:, jnp.array([0, 3, 2, 1])])
    ind1 = jnp.argsort(codes1)
    ind2 = jnp.argsort(codes2)
    inv1 = jnp.zeros((n,), ind1.dtype).at[ind1].set(
        jnp.arange(n, dtype=ind1.dtype))
    ind12 = inv1[ind2]

    pts_pad = jnp.pad(point_coords, ((0, 0), (0, _C - 3)))

    x1 = _sc_gather(voxel_features, ind1)
    p1 = _sc_gather(pts_pad, ind1)[:, :16]
    p2 = _sc_gather(pts_pad, ind2)[:, :16]
    y0 = _voxformer(x1, p1, params[0])
    x2 = _sc_gather(y0, ind12)
    return _voxformer(x2, p2, params[1])
